# baseline (speedup 1.0000x reference)
import functools

import jax
import jax.numpy as jnp
from jax import lax
from jax.experimental import pallas as pl
from jax.experimental.pallas import tpu as pltpu
from jax.experimental.pallas import tpu_sc as plsc


def _sc_gather(tables, idx_t, n_tokens, n_slot, embed, n_workers, tok_chunk):
    per_w_tok = n_tokens // n_workers
    iters = per_w_tok // tok_chunk
    row_d = n_slot * embed
    n_tab = tables.shape[0]

    mesh = plsc.VectorSubcoreMesh(core_axis_name="c", subcore_axis_name="s")

    @functools.partial(
        pl.kernel,
        out_type=jax.ShapeDtypeStruct((n_tokens, row_d), jnp.float32),
        mesh=mesh,
        scratch_types=[
            pltpu.VMEM((n_slot, tok_chunk), jnp.int32),
            pltpu.VMEM((n_slot * tok_chunk, embed), jnp.float32),
            pltpu.SemaphoreType.DMA,
            pltpu.SemaphoreType.DMA,
        ],
        compiler_params=pltpu.CompilerParams(use_tc_tiling_on_sc=False),
    )
    def k(idx_hbm, tab_hbm, out_hbm, idx_v, rows_v, gsem, ssem):
        n_cores = 2
        wid = lax.axis_index("s") * n_cores + lax.axis_index("c")
        base_chunk = wid * iters

        def body(i, carry):
            chunk_id = base_chunk + i
            tok0 = chunk_id * tok_chunk
            pltpu.sync_copy(idx_hbm.at[pl.ds(chunk_id * n_slot, n_slot)], idx_v)

            def fire_gather(j, c):
                field = jnp.where(j < n_tab, j, j - n_tab)
                pltpu.async_copy(
                    tab_hbm.at[field].at[idx_v.at[j]],
                    rows_v.at[pl.ds(j * tok_chunk, tok_chunk)],
                    gsem,
                )
                return c

            lax.fori_loop(0, n_slot, fire_gather, 0)
            pltpu.make_async_copy(
                tab_hbm.at[0].at[pl.ds(0, n_slot * tok_chunk)], rows_v, gsem
            ).wait()

            def fire_store(j, c):
                pltpu.async_copy(
                    rows_v.at[pl.ds(j * tok_chunk, tok_chunk)],
                    out_hbm.at[pl.ds(tok0, tok_chunk), pl.ds(j * embed, embed)],
                    ssem,
                )
                return c

            lax.fori_loop(0, n_slot, fire_store, 0)
            pltpu.make_async_copy(
                tab_hbm.at[0].at[pl.ds(0, n_slot * tok_chunk)], rows_v, ssem
            ).wait()
            return carry

        lax.fori_loop(0, iters, body, 0)

    return k(idx_t, tables)


def _tc_tail(ga, gb, num2d, w_a, w_b, w_num, b_num, w_tail, b_final, block_t):
    t, da = ga.shape
    db = gb.shape[1]
    num_dim = num2d.shape[1]
    embed = w_num.shape[1]
    d_model = w_tail.shape[1]
    grid = (t // block_t,)

    def body(ga_ref, gb_ref, n_ref, wa_ref, wb_ref, wn_ref, bn_ref, wt_ref,
             bf_ref, o_ref):
        nump = (
            jnp.dot(n_ref[...], wn_ref[...], preferred_element_type=jnp.float32)
            + bn_ref[...]
        )
        o_ref[...] = (
            jnp.dot(ga_ref[...], wa_ref[...], preferred_element_type=jnp.float32)
            + jnp.dot(gb_ref[...], wb_ref[...], preferred_element_type=jnp.float32)
            + jnp.dot(nump, wt_ref[...], preferred_element_type=jnp.float32)
            + bf_ref[...]
        )

    return pl.pallas_call(
        body,
        grid=grid,
        in_specs=[
            pl.BlockSpec((block_t, da), lambda i: (i, 0)),
            pl.BlockSpec((block_t, db), lambda i: (i, 0)),
            pl.BlockSpec((block_t, num_dim), lambda i: (i, 0)),
            pl.BlockSpec((da, d_model), lambda i: (0, 0)),
            pl.BlockSpec((db, d_model), lambda i: (0, 0)),
            pl.BlockSpec((num_dim, embed), lambda i: (0, 0)),
            pl.BlockSpec((1, embed), lambda i: (0, 0)),
            pl.BlockSpec((embed, d_model), lambda i: (0, 0)),
            pl.BlockSpec((1, d_model), lambda i: (0, 0)),
        ],
        out_specs=pl.BlockSpec((block_t, d_model), lambda i: (i, 0)),
        out_shape=jax.ShapeDtypeStruct((t, d_model), jnp.float32),
    )(
        ga,
        gb,
        num2d,
        w_a,
        w_b,
        w_num,
        b_num.reshape(1, embed),
        w_tail,
        b_final.reshape(1, d_model),
    )


def kernel(cat_feats, num_feats, tables, W_num, b_num, W_final, b_final):
    b, l, n_cat = cat_feats.shape
    _, vocab, embed = tables.shape
    num_dim = num_feats.shape[-1]
    d_model = W_final.shape[1]
    t = b * l
    tok_chunk = 128
    n_a = 16
    n_b = n_cat - n_a
    n_slot_b = n_b + 2

    cat_t = (
        cat_feats.reshape(t // tok_chunk, tok_chunk, n_cat)
        .astype(jnp.int32)
        .transpose(0, 2, 1)
    )
    idx_a = cat_t[:, :n_a, :].reshape(t // tok_chunk * n_a, tok_chunk)
    idx_b = jnp.concatenate(
        [cat_t[:, n_a:, :], cat_t[:, :2, :]], axis=1
    ).reshape(t // tok_chunk * n_slot_b, tok_chunk)

    g_a = _sc_gather(
        tables[:n_a], idx_a, t, n_a, embed, n_workers=32, tok_chunk=tok_chunk
    )
    g_b = _sc_gather(
        tables[n_a:], idx_b, t, n_slot_b, embed, n_workers=32,
        tok_chunk=tok_chunk,
    )

    cat_d = n_cat * embed
    w_a = W_final[: n_a * embed]
    w_b = jnp.zeros((n_slot_b * embed, d_model), jnp.float32).at[
        : n_b * embed
    ].set(W_final[n_a * embed : cat_d])
    w_tail = W_final[cat_d:]

    out = _tc_tail(
        g_a, g_b, num_feats.reshape(t, num_dim), w_a, w_b, W_num, b_num,
        w_tail, b_final, block_t=2048,
    )
    return out.reshape(b, l, d_model)
